# Initial kernel scaffold; baseline (speedup 1.0000x reference)
import jax
import jax.numpy as jnp
from jax.experimental import pallas as pl


def kernel(noisy_data, data, condition_mask):
    raise NotImplementedError("write your pallas kernel here")


# speedup vs baseline: 29.1504x; 29.1504x over previous
import functools

import jax
import jax.numpy as jnp
from jax.experimental import pallas as pl


QBLK = 256


def _knn_body(q_ref, k_ref, m_ref, md_ref, idx_ref, *, ns):
    i = pl.program_id(1)
    q = q_ref[0]
    k = k_ref[0]
    m = m_ref[...] != 0
    qd = k_ref[0, pl.ds(i * QBLK, QBLK), :]
    q = jnp.where(m, qd, q)

    qn = jnp.sum(q * q, axis=1)
    kn = jnp.sum(k * k, axis=1)
    dots = jax.lax.dot_general(
        k, q, (((1,), (1,)), ((), ())),
        precision=jax.lax.Precision.HIGHEST,
        preferred_element_type=jnp.float32,
    )
    d2 = jnp.maximum(kn[:, None] - 2.0 * dots + qn[None, :], 0.0)

    md = jnp.min(d2, axis=0, keepdims=True)
    ids = jax.lax.broadcasted_iota(jnp.int32, d2.shape, 0)
    amin = jnp.min(jnp.where(d2 == md, ids, ns), axis=0, keepdims=True)

    md_ref[0, :, pl.ds(i * QBLK, QBLK)] = jnp.sqrt(md)
    idx_ref[0, :, pl.ds(i * QBLK, QBLK)] = amin


def kernel(noisy_data, data, condition_mask):
    B, NS, D = noisy_data.shape
    mask_i32 = condition_mask.astype(jnp.int32).reshape(1, D)
    md, idx = pl.pallas_call(
        functools.partial(_knn_body, ns=NS),
        grid=(B, NS // QBLK),
        in_specs=[
            pl.BlockSpec((1, QBLK, D), lambda b, i: (b, i, 0)),
            pl.BlockSpec((1, NS, D), lambda b, i: (b, 0, 0)),
            pl.BlockSpec((1, D), lambda b, i: (0, 0)),
        ],
        out_specs=[
            pl.BlockSpec((1, 1, NS), lambda b, i: (b, 0, 0)),
            pl.BlockSpec((1, 1, NS), lambda b, i: (b, 0, 0)),
        ],
        out_shape=[
            jax.ShapeDtypeStruct((B, 1, NS), jnp.float32),
            jax.ShapeDtypeStruct((B, 1, NS), jnp.int32),
        ],
    )(noisy_data, data, mask_i32)
    return md.reshape(B, NS), idx.reshape(B, NS)


# speedup vs baseline: 33.4408x; 1.1472x over previous
import functools

import jax
import jax.numpy as jnp
from jax.experimental import pallas as pl


QBLK = 256


def _knn_body(q_ref, k_ref, m_ref, md_ref, idx_ref, *, ns):
    i = pl.program_id(1)
    q = q_ref[0]
    k = k_ref[0]
    m = m_ref[...] != 0
    qd = k_ref[0, pl.ds(i * QBLK, QBLK), :]
    q = jnp.where(m, qd, q)

    qn = jnp.sum(q * q, axis=1)
    kn = jnp.sum(k * k, axis=1)
    dots = jax.lax.dot_general(
        k * -2.0, q, (((1,), (1,)), ((), ())),
        precision=jax.lax.Precision.HIGHEST,
        preferred_element_type=jnp.float32,
    )
    e = dots + kn[:, None]

    me = jnp.min(e, axis=0, keepdims=True)
    ids = jax.lax.broadcasted_iota(jnp.int32, e.shape, 0)
    amin = jnp.min(jnp.where(e == me, ids, ns), axis=0, keepdims=True)

    d2 = jnp.maximum(me + qn.reshape(1, -1), 0.0)
    md_ref[0, :, pl.ds(i * QBLK, QBLK)] = jnp.sqrt(d2)
    idx_ref[0, :, pl.ds(i * QBLK, QBLK)] = amin


def kernel(noisy_data, data, condition_mask):
    B, NS, D = noisy_data.shape
    mask_i32 = condition_mask.astype(jnp.int32).reshape(1, D)
    md, idx = pl.pallas_call(
        functools.partial(_knn_body, ns=NS),
        grid=(B, NS // QBLK),
        in_specs=[
            pl.BlockSpec((1, QBLK, D), lambda b, i: (b, i, 0)),
            pl.BlockSpec((1, NS, D), lambda b, i: (b, 0, 0)),
            pl.BlockSpec((1, D), lambda b, i: (0, 0)),
        ],
        out_specs=[
            pl.BlockSpec((1, 1, NS), lambda b, i: (b, 0, 0)),
            pl.BlockSpec((1, 1, NS), lambda b, i: (b, 0, 0)),
        ],
        out_shape=[
            jax.ShapeDtypeStruct((B, 1, NS), jnp.float32),
            jax.ShapeDtypeStruct((B, 1, NS), jnp.int32),
        ],
    )(noisy_data, data, mask_i32)
    return md.reshape(B, NS), idx.reshape(B, NS)


# speedup vs baseline: 55.3293x; 1.6545x over previous
import functools

import jax
import jax.numpy as jnp
from jax.experimental import pallas as pl


QBLK = 256


def _knn_body(q_ref, k_ref, m_ref, md_ref, idx_ref, *, ns):
    i = pl.program_id(1)
    q = q_ref[0]
    k = k_ref[0]
    m = m_ref[...] != 0
    qd = k_ref[0, pl.ds(i * QBLK, QBLK), :]
    q = jnp.where(m, qd, q)

    qn = jnp.sum(q * q, axis=1)
    kn = jnp.sum(k * k, axis=1)
    ksc = k * -2.0
    kh = ksc.astype(jnp.bfloat16)
    kl = (ksc - kh.astype(jnp.float32)).astype(jnp.bfloat16)
    qh = q.astype(jnp.bfloat16)
    ql = (q - qh.astype(jnp.float32)).astype(jnp.bfloat16)
    kcat = jnp.concatenate([kh, kh, kl], axis=1)
    qcat = jnp.concatenate([qh, ql, qh], axis=1)
    dots = jax.lax.dot_general(
        kcat, qcat, (((1,), (1,)), ((), ())),
        preferred_element_type=jnp.float32,
    )
    e = dots + kn[:, None]

    me = jnp.min(e, axis=0, keepdims=True)
    ids = jax.lax.broadcasted_iota(jnp.int32, e.shape, 0)
    amin = jnp.min(jnp.where(e == me, ids, ns), axis=0, keepdims=True)

    d2 = jnp.maximum(me + qn.reshape(1, -1), 0.0)
    md_ref[0, :, pl.ds(i * QBLK, QBLK)] = jnp.sqrt(d2)
    idx_ref[0, :, pl.ds(i * QBLK, QBLK)] = amin


def kernel(noisy_data, data, condition_mask):
    B, NS, D = noisy_data.shape
    mask_i32 = condition_mask.astype(jnp.int32).reshape(1, D)
    md, idx = pl.pallas_call(
        functools.partial(_knn_body, ns=NS),
        grid=(B, NS // QBLK),
        in_specs=[
            pl.BlockSpec((1, QBLK, D), lambda b, i: (b, i, 0)),
            pl.BlockSpec((1, NS, D), lambda b, i: (b, 0, 0)),
            pl.BlockSpec((1, D), lambda b, i: (0, 0)),
        ],
        out_specs=[
            pl.BlockSpec((1, 1, NS), lambda b, i: (b, 0, 0)),
            pl.BlockSpec((1, 1, NS), lambda b, i: (b, 0, 0)),
        ],
        out_shape=[
            jax.ShapeDtypeStruct((B, 1, NS), jnp.float32),
            jax.ShapeDtypeStruct((B, 1, NS), jnp.int32),
        ],
    )(noisy_data, data, mask_i32)
    return md.reshape(B, NS), idx.reshape(B, NS)


# speedup vs baseline: 64.6629x; 1.1687x over previous
import functools

import jax
import jax.numpy as jnp
from jax.experimental import pallas as pl


QBLK = 512


def _knn_body(q_ref, k_ref, m_ref, md_ref, idx_ref, *, ns):
    i = pl.program_id(1)
    q = q_ref[0]
    k = k_ref[0]
    m = m_ref[...] != 0
    qd = k_ref[0, pl.ds(i * QBLK, QBLK), :]
    q = jnp.where(m, qd, q)

    qn = jnp.sum(q * q, axis=1)
    kn = jnp.sum(k * k, axis=1)
    ksc = k * -2.0
    kh = ksc.astype(jnp.bfloat16)
    kl = (ksc - kh.astype(jnp.float32)).astype(jnp.bfloat16)
    qh = q.astype(jnp.bfloat16)
    ql = (q - qh.astype(jnp.float32)).astype(jnp.bfloat16)
    kcat = jnp.concatenate([kh, kh, kl], axis=1)
    qcat = jnp.concatenate([qh, ql, qh], axis=1)
    dots = jax.lax.dot_general(
        kcat, qcat, (((1,), (1,)), ((), ())),
        preferred_element_type=jnp.float32,
    )
    e = dots + kn[:, None]

    me = jnp.min(e, axis=0, keepdims=True)
    ids = jax.lax.broadcasted_iota(jnp.int32, e.shape, 0)
    amin = jnp.min(jnp.where(e == me, ids, ns), axis=0, keepdims=True)

    d2 = jnp.maximum(me + qn.reshape(1, -1), 0.0)
    md_ref[0, :, pl.ds(i * QBLK, QBLK)] = jnp.sqrt(d2)
    idx_ref[0, :, pl.ds(i * QBLK, QBLK)] = amin


def kernel(noisy_data, data, condition_mask):
    B, NS, D = noisy_data.shape
    mask_i32 = condition_mask.astype(jnp.int32).reshape(1, D)
    md, idx = pl.pallas_call(
        functools.partial(_knn_body, ns=NS),
        grid=(B, NS // QBLK),
        in_specs=[
            pl.BlockSpec((1, QBLK, D), lambda b, i: (b, i, 0)),
            pl.BlockSpec((1, NS, D), lambda b, i: (b, 0, 0)),
            pl.BlockSpec((1, D), lambda b, i: (0, 0)),
        ],
        out_specs=[
            pl.BlockSpec((1, 1, NS), lambda b, i: (b, 0, 0)),
            pl.BlockSpec((1, 1, NS), lambda b, i: (b, 0, 0)),
        ],
        out_shape=[
            jax.ShapeDtypeStruct((B, 1, NS), jnp.float32),
            jax.ShapeDtypeStruct((B, 1, NS), jnp.int32),
        ],
    )(noisy_data, data, mask_i32)
    return md.reshape(B, NS), idx.reshape(B, NS)


# speedup vs baseline: 82.7077x; 1.2791x over previous
import functools

import jax
import jax.numpy as jnp
from jax.experimental import pallas as pl


QBLK = 2048


def _knn_body(q_ref, k_ref, m_ref, md_ref, idx_ref, *, ns):
    i = pl.program_id(1)
    q = q_ref[0]
    k = k_ref[0]
    m = m_ref[...] != 0
    qd = k_ref[0, pl.ds(i * QBLK, QBLK), :]
    q = jnp.where(m, qd, q)

    qn = jnp.sum(q * q, axis=1)
    kn = jnp.sum(k * k, axis=1)
    ksc = k * -2.0
    kh = ksc.astype(jnp.bfloat16)
    kl = (ksc - kh.astype(jnp.float32)).astype(jnp.bfloat16)
    qh = q.astype(jnp.bfloat16)
    ql = (q - qh.astype(jnp.float32)).astype(jnp.bfloat16)
    kcat = jnp.concatenate([kh, kh, kl], axis=1)
    qcat = jnp.concatenate([qh, ql, qh], axis=1)
    dots = jax.lax.dot_general(
        kcat, qcat, (((1,), (1,)), ((), ())),
        preferred_element_type=jnp.float32,
    )
    e = dots + kn[:, None]

    me = jnp.min(e, axis=0, keepdims=True)
    ids = jax.lax.broadcasted_iota(jnp.int32, e.shape, 0)
    amin = jnp.min(jnp.where(e == me, ids, ns), axis=0, keepdims=True)

    d2 = jnp.maximum(me + qn.reshape(1, -1), 0.0)
    md_ref[0, :, pl.ds(i * QBLK, QBLK)] = jnp.sqrt(d2)
    idx_ref[0, :, pl.ds(i * QBLK, QBLK)] = amin


def kernel(noisy_data, data, condition_mask):
    B, NS, D = noisy_data.shape
    mask_i32 = condition_mask.astype(jnp.int32).reshape(1, D)
    md, idx = pl.pallas_call(
        functools.partial(_knn_body, ns=NS),
        grid=(B, NS // QBLK),
        in_specs=[
            pl.BlockSpec((1, QBLK, D), lambda b, i: (b, i, 0)),
            pl.BlockSpec((1, NS, D), lambda b, i: (b, 0, 0)),
            pl.BlockSpec((1, D), lambda b, i: (0, 0)),
        ],
        out_specs=[
            pl.BlockSpec((1, 1, NS), lambda b, i: (b, 0, 0)),
            pl.BlockSpec((1, 1, NS), lambda b, i: (b, 0, 0)),
        ],
        out_shape=[
            jax.ShapeDtypeStruct((B, 1, NS), jnp.float32),
            jax.ShapeDtypeStruct((B, 1, NS), jnp.int32),
        ],
    )(noisy_data, data, mask_i32)
    return md.reshape(B, NS), idx.reshape(B, NS)


# speedup vs baseline: 85.4135x; 1.0327x over previous
import functools

import jax
import jax.numpy as jnp
from jax.experimental import pallas as pl


QBLK = 2048


def _knn_body(q_ref, k_ref, m_ref, md_ref, idx_ref, *, ns):
    i = pl.program_id(1)
    q = q_ref[0]
    k = k_ref[0]
    m = m_ref[...] != 0
    qd = k_ref[0, pl.ds(i * QBLK, QBLK), :]
    q = jnp.where(m, qd, q)

    qn = jnp.sum(q * q, axis=1)
    kn = jnp.sum(k * k, axis=1)
    ksc = k * -2.0
    kh = ksc.astype(jnp.bfloat16)
    kl = (ksc - kh.astype(jnp.float32)).astype(jnp.bfloat16)
    qh = q.astype(jnp.bfloat16)
    ql = (q - qh.astype(jnp.float32)).astype(jnp.bfloat16)
    kcat = jnp.concatenate([kh, kh, kl], axis=1)
    qcat = jnp.concatenate([qh, ql, qh], axis=1)
    dots = jax.lax.dot_general(
        kcat, qcat, (((1,), (1,)), ((), ())),
        preferred_element_type=jnp.float32,
    )
    e = dots + kn[:, None]

    me = jnp.min(e, axis=0, keepdims=True)
    eq = e == me
    maskb = eq.astype(jnp.bfloat16)
    j3 = jax.lax.broadcasted_iota(jnp.int32, (3, ns), 1)
    r3 = jax.lax.broadcasted_iota(jnp.int32, (3, ns), 0)
    w = jnp.where(r3 == 0, j3 >> 8, jnp.where(r3 == 1, j3 & 255, 1))
    digits = jax.lax.dot_general(
        w.astype(jnp.bfloat16), maskb, (((1,), (0,)), ((), ())),
        preferred_element_type=jnp.float32,
    )
    idx_fast = (digits[0:1] * 256.0 + digits[1:2]).astype(jnp.int32)
    idx_ref[0, :, pl.ds(i * QBLK, QBLK)] = idx_fast

    @pl.when(jnp.max(digits[2:3]) > 1.5)
    def _tie_fallback():
        ids = jax.lax.broadcasted_iota(jnp.int32, e.shape, 0)
        amin = jnp.min(jnp.where(eq, ids, ns), axis=0, keepdims=True)
        idx_ref[0, :, pl.ds(i * QBLK, QBLK)] = amin

    d2 = jnp.maximum(me + qn.reshape(1, -1), 0.0)
    md_ref[0, :, pl.ds(i * QBLK, QBLK)] = jnp.sqrt(d2)


def kernel(noisy_data, data, condition_mask):
    B, NS, D = noisy_data.shape
    mask_i32 = condition_mask.astype(jnp.int32).reshape(1, D)
    md, idx = pl.pallas_call(
        functools.partial(_knn_body, ns=NS),
        grid=(B, NS // QBLK),
        in_specs=[
            pl.BlockSpec((1, QBLK, D), lambda b, i: (b, i, 0)),
            pl.BlockSpec((1, NS, D), lambda b, i: (b, 0, 0)),
            pl.BlockSpec((1, D), lambda b, i: (0, 0)),
        ],
        out_specs=[
            pl.BlockSpec((1, 1, NS), lambda b, i: (b, 0, 0)),
            pl.BlockSpec((1, 1, NS), lambda b, i: (b, 0, 0)),
        ],
        out_shape=[
            jax.ShapeDtypeStruct((B, 1, NS), jnp.float32),
            jax.ShapeDtypeStruct((B, 1, NS), jnp.int32),
        ],
    )(noisy_data, data, mask_i32)
    return md.reshape(B, NS), idx.reshape(B, NS)
